# baseline (speedup 1.0000x reference)
import functools

import jax
import jax.numpy as jnp
from jax import lax
from jax.experimental import pallas as pl
from jax.experimental.pallas import tpu as pltpu
from jax.experimental.pallas import tpu_sc as plsc

N = 100000
E = 1600000
G = 1000
BN_EPS = 1e-5

NC, NS, L = 2, 16, 16
NP = 102400
EP = 1622016
EPR = EP // 128
CH = 384
NSUB = 3
CHUNKS = EP // (NS * CH)
ACC_R = 100352
ZPT = ACC_R // NS
PG_R = 1008
PW = 48
PCH = 25
PB = NP // (NC * NS)

_mesh = plsc.VectorSubcoreMesh(
    core_axis_name="c", subcore_axis_name="s", num_cores=NC, num_subcores=NS)
_sc_params = pltpu.CompilerParams(use_tc_tiling_on_sc=False)


@functools.partial(
    pl.kernel,
    out_type=jax.ShapeDtypeStruct((NC, NP, L), jnp.float32),
    mesh=_mesh,
    compiler_params=_sc_params,
    scratch_types=(
        [pltpu.VMEM_SHARED((ACC_R, L), jnp.float32)]
        + [pltpu.VMEM((3 * NSUB, 128), jnp.int32) for _ in range(4)]
        + [pltpu.VMEM((CH, L), jnp.float32) for _ in range(2)]
        + [pltpu.VMEM((CH, L), jnp.float32) for _ in range(2)]
        + [pltpu.SemaphoreType.DMA for _ in range(8)]
    ),
)
def _edge_pass(h2_hbm, sd_hbm, e_hbm, agg_hbm, acc_sh,
               sd0, sd1, sd2, sd3, ev0, ev1,
               hr0, hr1,
               si0, si1, si2, si3, sg0, sg1, ss0, ss1):
    c = lax.axis_index("c")
    s = lax.axis_index("s")
    sd = [sd0, sd1, sd2, sd3]
    ebuf = [ev0, ev1]
    hrows = [hr0, hr1]
    semI = [si0, si1, si2, si3]
    semG = [sg0, sg1]
    semS = [ss0, ss1]

    def _zrow(b, _):
        hr0[b, :] = jnp.zeros((L,), jnp.float32)
        return ()
    lax.fori_loop(0, CH, _zrow, (), unroll=8)
    z0 = s * ZPT
    zcps = []
    for k in range(ZPT // CH):
        zcps.append(pltpu.async_copy(hr0, acc_sh.at[pl.ds(z0 + k * CH, CH)],
                                     si0))
    rem = ZPT % CH
    if rem:
        zcps.append(pltpu.async_copy(
            hr0.at[pl.ds(0, rem)],
            acc_sh.at[pl.ds(z0 + (ZPT // CH) * CH, rem)], si0))
    for cp in zcps:
        cp.wait()
    plsc.subcore_barrier()

    def fire_loads(g, q):
        t = s * CHUNKS + g
        pltpu.async_copy(sd_hbm.at[c, pl.ds(t * 3 * NSUB, 3 * NSUB)],
                         sd[q], semI[q])

    def wait_loads(q):
        pltpu.make_async_copy(sd_hbm.at[0, pl.ds(0, 3 * NSUB)], sd[q],
                              semI[q]).wait()

    def fire_gather(g, p, q):
        for j in range(NSUB):
            pltpu.async_copy(h2_hbm.at[sd[q].at[j]],
                             hrows[p].at[pl.ds(j * 128, 128)], semG[p])
        for j in range(NSUB):
            pltpu.async_copy(e_hbm.at[sd[q].at[2 * NSUB + j]],
                             ebuf[p].at[pl.ds(j * 128, 128)], semG[p])

    def wait_gather(p):
        pltpu.make_async_copy(h2_hbm.at[pl.ds(0, CH)], hrows[p],
                              semG[p]).wait()
        pltpu.make_async_copy(e_hbm.at[pl.ds(0, CH)], ebuf[p],
                              semG[p]).wait()

    def compute(p, q):
        hv = hrows[p]
        ev = ebuf[p]

        def _edge(b, _):
            hv[b, :] = jnp.maximum(hv[b, :] + ev[b, :], 0.0)
            return ()
        lax.fori_loop(0, CH, _edge, (), unroll=8)

    def fire_scatter(p, q):
        for j in range(NSUB):
            pltpu.async_copy(hrows[p].at[pl.ds(j * 128, 128)],
                             acc_sh.at[sd[q].at[NSUB + j]], semS[p], add=True)

    def wait_scatter(p):
        pltpu.make_async_copy(hrows[p], acc_sh.at[pl.ds(0, CH)],
                              semS[p]).wait()

    def phase(cidx, u, do_loads=True):
        wait_scatter(u % 2)
        wait_loads((u + 2) % 4)
        fire_gather(cidx, u % 2, (u + 2) % 4)
        wait_gather((u + 1) % 2)
        compute((u + 1) % 2, (u + 1) % 4)
        fire_scatter((u + 1) % 2, (u + 1) % 4)
        if do_loads:
            fire_loads(cidx + 2, u % 4)

    for q in range(4):
        fire_loads(q, q)
    wait_loads(0)
    fire_gather(0, 0, 0)
    wait_loads(1)
    fire_gather(1, 1, 1)
    wait_gather(0)
    compute(0, 0)
    fire_scatter(0, 0)

    def _body(k, _):
        cbase = 4 * k + 2
        for u in range(4):
            phase(cbase + u, u)
        return ()
    lax.fori_loop(0, (CHUNKS - 4) // 4, _body, ())

    phase(CHUNKS - 2, 0, do_loads=False)
    phase(CHUNKS - 1, 1, do_loads=False)
    wait_gather(1)
    compute(1, 3)
    fire_scatter(1, 3)
    wait_scatter(0)
    wait_scatter(1)
    plsc.subcore_barrier()

    w0 = s * ZPT
    nwb = ZPT // CH + (1 if ZPT % CH else 0)

    def _wb(k):
        off = k * CH
        size = min(CH, ZPT - off)
        return off, size

    def fire_rd(k):
        off, size = _wb(k)
        pltpu.async_copy(acc_sh.at[pl.ds(w0 + off, size)],
                         hrows[k % 2].at[pl.ds(0, size)], semG[k % 2])

    def wait_rd(k):
        off, size = _wb(k)
        pltpu.make_async_copy(acc_sh.at[pl.ds(0, size)],
                              hrows[k % 2].at[pl.ds(0, size)],
                              semG[k % 2]).wait()

    def fire_wr(k):
        off, size = _wb(k)
        pltpu.async_copy(hrows[k % 2].at[pl.ds(0, size)],
                         agg_hbm.at[c, pl.ds(w0 + off, size)], semS[k % 2])

    def wait_wr(k):
        off, size = _wb(k)
        pltpu.make_async_copy(hrows[k % 2].at[pl.ds(0, size)],
                              agg_hbm.at[c, pl.ds(0, size)],
                              semS[k % 2]).wait()

    fire_rd(0)
    for k in range(nwb):
        if k + 1 < nwb:
            if k - 1 >= 0:
                wait_wr(k - 1)
            fire_rd(k + 1)
        wait_rd(k)
        fire_wr(k)
    for k in (nwb - 2, nwb - 1):
        if k >= 0:
            wait_wr(k)


@functools.partial(
    pl.kernel,
    out_type=jax.ShapeDtypeStruct((NC, PG_R, PW), jnp.float32),
    mesh=_mesh,
    compiler_params=_sc_params,
    scratch_types=[
        pltpu.VMEM_SHARED((PG_R, PW), jnp.float32),
        pltpu.VMEM((128, 32), jnp.float32),
        pltpu.VMEM((128, PW), jnp.float32),
        pltpu.VMEM((PCH, 128), jnp.int32),
        pltpu.VMEM((2, 128), jnp.float32),
    ],
)
def _pool_pass(y_hbm, bidx_hbm, stats_hbm, pooled_hbm,
               acc_sh, yv, mbuf, bidx_v, ssv):
    c = lax.axis_index("c")
    s = lax.axis_index("s")
    w = s * NC + c
    zpt = PG_R // NS

    pltpu.sync_copy(stats_hbm.at[pl.ds(2, 2)], ssv)
    sc0 = ssv[0, 0:16]
    sc1 = ssv[0, 16:32]
    sh0 = ssv[1, 0:16]
    sh1 = ssv[1, 16:32]

    zv = jnp.zeros((L,), jnp.float32)
    def _zrow(b, _):
        mbuf[b, 0:16] = zv
        mbuf[b, 16:32] = zv
        mbuf[b, 32:48] = zv
        return ()
    lax.fori_loop(0, 128, _zrow, (), unroll=4)
    pltpu.sync_copy(mbuf.at[pl.ds(0, zpt)], acc_sh.at[pl.ds(s * zpt, zpt)])

    cvec = jnp.where(lax.iota(jnp.int32, L) == 0, 1.0, 0.0).astype(jnp.float32)
    def _crow(b, _):
        mbuf[b, 32:48] = cvec
        return ()
    lax.fori_loop(0, 128, _crow, ())
    plsc.subcore_barrier()

    pltpu.sync_copy(bidx_hbm.at[pl.ds(w * PCH, PCH)], bidx_v)

    def _chunk(t, _):
        r0 = w * PB + t * 128
        pltpu.sync_copy(y_hbm.at[pl.ds(r0, 128)], yv)

        def _row(b, _):
            mbuf[b, 0:16] = jnp.maximum(yv[b, 0:16] * sc0 + sh0, 0.0)
            mbuf[b, 16:32] = jnp.maximum(yv[b, 16:32] * sc1 + sh1, 0.0)
            return ()
        lax.fori_loop(0, 128, _row, (), unroll=4)

        pltpu.sync_copy(mbuf, acc_sh.at[bidx_v.at[t]], add=True)
        return ()

    lax.fori_loop(0, PCH, _chunk, ())
    plsc.subcore_barrier()

    pltpu.sync_copy(acc_sh.at[pl.ds(s * zpt, zpt)], mbuf.at[pl.ds(0, zpt)])
    pltpu.sync_copy(mbuf.at[pl.ds(0, zpt)],
                    pooled_hbm.at[c, pl.ds(s * zpt, zpt)])


_BN1 = 2048
_NBLK = NP // _BN1


def _k1_body(x_ref, w_ref, b_ref, out_ref):
    y = jnp.dot(x_ref[...], w_ref[...], preferred_element_type=jnp.float32,
                precision=lax.Precision.HIGHEST) + b_ref[...]
    out_ref[0] = y[:, :16]
    out_ref[1] = y[:, 16:]


def _node_mlp(xp, node_w, node_b):
    return pl.pallas_call(
        _k1_body,
        grid=(_NBLK,),
        in_specs=[
            pl.BlockSpec((_BN1, 14), lambda i: (i, 0)),
            pl.BlockSpec((14, 32), lambda i: (0, 0)),
            pl.BlockSpec((32,), lambda i: (0,)),
        ],
        out_specs=pl.BlockSpec((2, _BN1, 16), lambda i: (0, i, 0)),
        out_shape=jax.ShapeDtypeStruct((2, NP, 16), jnp.float32),
    )(xp, node_w, node_b)


_EBLK = 4096


def _edge_mlp(attrp, edge_w, edge_b):
    return pl.pallas_call(
        _k1_body,
        grid=(EP // _EBLK,),
        in_specs=[
            pl.BlockSpec((_EBLK, 3), lambda i: (i, 0)),
            pl.BlockSpec((3, 32), lambda i: (0, 0)),
            pl.BlockSpec((32,), lambda i: (0,)),
        ],
        out_specs=pl.BlockSpec((2, _EBLK, 16), lambda i: (0, i, 0)),
        out_shape=jax.ShapeDtypeStruct((2, EP, 16), jnp.float32),
    )(attrp, edge_w, edge_b)


def _k2_body(h_ref, agg_ref, w1_ref, b1_ref, w2_ref, b2_ref, g_ref, bb_ref,
             y_ref, st_ref):
    i = pl.program_id(0)
    h = jnp.concatenate([h_ref[0], h_ref[1]], axis=1)
    a = jnp.concatenate([agg_ref[0], agg_ref[1]], axis=1)
    z = h + a
    t = jnp.maximum(jnp.dot(z, w1_ref[...], preferred_element_type=jnp.float32,
                            precision=lax.Precision.HIGHEST) + b1_ref[...], 0.0)
    y = jnp.dot(t, w2_ref[...], preferred_element_type=jnp.float32,
                precision=lax.Precision.HIGHEST) + b2_ref[...]
    y_ref[...] = y

    rows = lax.broadcasted_iota(jnp.int32, (_BN1, 1), 0) + i * _BN1
    ym = jnp.where(rows < N, y, 0.0)

    @pl.when(i == 0)
    def _():
        st_ref[...] = jnp.zeros((8, 128), jnp.float32)

    ssum = jnp.sum(ym, axis=0)
    ssq = jnp.sum(ym * ym, axis=0)
    delta = jnp.pad(jnp.stack([ssum, ssq], axis=0), ((0, 6), (0, 96)))
    st_ref[...] += delta

    @pl.when(i == _NBLK - 1)
    def _():
        st = st_ref[...]
        mu = st[0, :32] / N
        var = st[1, :32] / N - mu * mu
        scale = g_ref[...] * lax.rsqrt(var + BN_EPS)
        shift = bb_ref[...] - mu * scale
        st_ref[pl.ds(2, 2), :] = jnp.pad(
            jnp.stack([scale, shift], axis=0), ((0, 0), (0, 96)))


def _gine_mlp(h2, agg, w1, b1, w2, b2, g, bb):
    return pl.pallas_call(
        _k2_body,
        grid=(_NBLK,),
        in_specs=[
            pl.BlockSpec((2, _BN1, 16), lambda i: (0, i, 0)),
            pl.BlockSpec((2, _BN1, 16), lambda i: (0, i, 0)),
            pl.BlockSpec((32, 75), lambda i: (0, 0)),
            pl.BlockSpec((75,), lambda i: (0,)),
            pl.BlockSpec((75, 32), lambda i: (0, 0)),
            pl.BlockSpec((32,), lambda i: (0,)),
            pl.BlockSpec((32,), lambda i: (0,)),
            pl.BlockSpec((32,), lambda i: (0,)),
        ],
        out_specs=[
            pl.BlockSpec((_BN1, 32), lambda i: (i, 0)),
            pl.BlockSpec((8, 128), lambda i: (0, 0)),
        ],
        out_shape=[
            jax.ShapeDtypeStruct((NP, 32), jnp.float32),
            jax.ShapeDtypeStruct((8, 128), jnp.float32),
        ],
    )(h2, agg, w1, b1, w2, b2, g, bb)


def _k4_body(y_ref, st_ref, out_ref):
    sc = st_ref[2, :32]
    sh = st_ref[3, :32]
    hh = jnp.maximum(y_ref[...] * sc + sh, 0.0)
    out_ref[0] = hh[:, :16]
    out_ref[1] = hh[:, 16:]


def _bn_apply(y, stats):
    return pl.pallas_call(
        _k4_body,
        grid=(_NBLK,),
        in_specs=[
            pl.BlockSpec((_BN1, 32), lambda i: (i, 0)),
            pl.BlockSpec((8, 128), lambda i: (0, 0)),
        ],
        out_specs=pl.BlockSpec((2, _BN1, 16), lambda i: (0, i, 0)),
        out_shape=jax.ShapeDtypeStruct((2, NP, 16), jnp.float32),
    )(y, stats)


def _k5_body(p_ref, w1_ref, b1_ref, w2_ref, b2_ref, out_ref):
    sall = p_ref[0] + p_ref[1]
    sums = sall[:G, :32]
    cnt = sall[:G, 32:33]
    gx = sums / jnp.maximum(cnt, 1.0)
    t = jnp.maximum(jnp.dot(gx, w1_ref[...], preferred_element_type=jnp.float32,
                            precision=lax.Precision.HIGHEST) + b1_ref[...], 0.0)
    out_ref[...] = jnp.dot(t, w2_ref[...], preferred_element_type=jnp.float32,
                           precision=lax.Precision.HIGHEST) + b2_ref[...]


def _head(pooled, l1_w, l1_b, l2_w, l2_b):
    return pl.pallas_call(
        _k5_body,
        out_shape=jax.ShapeDtypeStruct((G, 2), jnp.float32),
    )(pooled, l1_w, l1_b, l2_w, l2_b)


def kernel(x, edge_index, edge_attr, batch,
           node_w, node_b, edge_w, edge_b,
           c0_w1, c0_b1, c0_w2, c0_b2, bn0_g, bn0_b,
           c1_w1, c1_b1, c1_w2, c1_b2, bn1_g, bn1_b,
           l1_w, l1_b, l2_w, l2_b):
    xp = jnp.pad(x, ((0, NP - N), (0, 0)))
    batchp = jnp.pad(batch, (0, NP - N), constant_values=G).reshape(
        NP // 128, 128)

    pad_e = EP - E
    ar = jnp.arange(pad_e, dtype=jnp.int32) % 256
    srcp = jnp.concatenate([edge_index[0], ar])
    src2 = jnp.stack([srcp, srcp + NP])
    dstp = jnp.concatenate([edge_index[1], N + ar])
    tc_ = EP // CH
    srcr = src2.reshape(2, tc_, NSUB, 128)
    dstr = jnp.broadcast_to(dstp.reshape(1, tc_, NSUB, 128),
                            (2, tc_, NSUB, 128))
    eidxr = jnp.arange(2 * EP, dtype=jnp.int32).reshape(2, tc_, NSUB, 128)
    sd2 = jnp.concatenate([srcr, dstr, eidxr], axis=2).reshape(
        2, tc_ * 3 * NSUB, 128)
    attrp = jnp.pad(edge_attr, ((0, pad_e), (0, 0)))

    e2 = _edge_mlp(attrp, edge_w, edge_b).reshape(2 * EP, 16)
    h2 = _node_mlp(xp, node_w, node_b)
    agg0 = _edge_pass(h2.reshape(2 * NP, 16), sd2, e2)
    y0, stats0 = _gine_mlp(h2, agg0, c0_w1, c0_b1, c0_w2, c0_b2, bn0_g, bn0_b)
    h2b = _bn_apply(y0, stats0)
    agg1 = _edge_pass(h2b.reshape(2 * NP, 16), sd2, e2)
    y1, stats1 = _gine_mlp(h2b, agg1, c1_w1, c1_b1, c1_w2, c1_b2, bn1_g, bn1_b)
    pooled = _pool_pass(y1, batchp, stats1)
    return _head(pooled, l1_w, l1_b, l2_w, l2_b)


# speedup vs baseline: 5.3925x; 5.3925x over previous
import functools

import jax
import jax.numpy as jnp
from jax import lax
from jax.experimental import pallas as pl
from jax.experimental.pallas import tpu as pltpu
from jax.experimental.pallas import tpu_sc as plsc

N = 100000
E = 1600000
G = 1000
BN_EPS = 1e-5

NC, NS, L = 2, 16, 16
NP = 102400
EP = 1605632
EPR = EP // 128
CH = 512
NSUB = 4
CHUNKS = EP // (NS * CH)
ACC_R = 100352
ZPT = ACC_R // NS
PG_R = 1008
PW = 48
PCH = 25
PB = NP // (NC * NS)

_mesh = plsc.VectorSubcoreMesh(
    core_axis_name="c", subcore_axis_name="s", num_cores=NC, num_subcores=NS)
_sc_params = pltpu.CompilerParams(use_tc_tiling_on_sc=False)


@functools.partial(
    pl.kernel,
    out_type=jax.ShapeDtypeStruct((NC, NP, L), jnp.float32),
    mesh=_mesh,
    compiler_params=_sc_params,
    scratch_types=(
        [pltpu.VMEM_SHARED((ACC_R, L), jnp.float32)]
        + [pltpu.VMEM((2 * NSUB, 128), jnp.int32) for _ in range(4)]
        + [pltpu.VMEM((3 * CH,), jnp.float32) for _ in range(4)]
        + [pltpu.VMEM((CH, L), jnp.float32) for _ in range(2)]
        + [pltpu.VMEM((4, L), jnp.float32)]
        + [pltpu.SemaphoreType.DMA for _ in range(8)]
    ),
)
def _edge_pass(h2_hbm, sd_hbm, a0_hbm, a1_hbm, a2_hbm, ewb_hbm, agg_hbm,
               acc_sh,
               sd0, sd1, sd2, sd3, at0, at1, at2, at3,
               hr0, hr1, ew_v,
               si0, si1, si2, si3, sg0, sg1, ss0, ss1):
    c = lax.axis_index("c")
    s = lax.axis_index("s")
    sd = [sd0, sd1, sd2, sd3]
    attr = [at0, at1, at2, at3]
    hrows = [hr0, hr1]
    semI = [si0, si1, si2, si3]
    semG = [sg0, sg1]
    semS = [ss0, ss1]

    pltpu.sync_copy(ewb_hbm.at[c], ew_v)

    def _zrow(b, _):
        hr0[b, :] = jnp.zeros((L,), jnp.float32)
        return ()
    lax.fori_loop(0, CH, _zrow, (), unroll=8)
    z0 = s * ZPT
    zcps = []
    for k in range(ZPT // CH):
        zcps.append(pltpu.async_copy(hr0, acc_sh.at[pl.ds(z0 + k * CH, CH)],
                                     si0))
    rem = ZPT % CH
    if rem:
        zcps.append(pltpu.async_copy(
            hr0.at[pl.ds(0, rem)],
            acc_sh.at[pl.ds(z0 + (ZPT // CH) * CH, rem)], si0))
    for cp in zcps:
        cp.wait()
    plsc.subcore_barrier()

    ew0 = ew_v[0, :]
    ew1 = ew_v[1, :]
    ew2 = ew_v[2, :]
    ebv = ew_v[3, :]

    def fire_loads(g, q):
        t = s * CHUNKS + g
        base = t * CH
        pltpu.async_copy(sd_hbm.at[c, pl.ds(t * 2 * NSUB, 2 * NSUB)],
                         sd[q], semI[q])
        pltpu.async_copy(a0_hbm.at[pl.ds(base, CH)],
                         attr[q].at[pl.ds(0, CH)], semI[q])
        pltpu.async_copy(a1_hbm.at[pl.ds(base, CH)],
                         attr[q].at[pl.ds(CH, CH)], semI[q])
        pltpu.async_copy(a2_hbm.at[pl.ds(base, CH)],
                         attr[q].at[pl.ds(2 * CH, CH)], semI[q])

    def wait_loads(q):
        pltpu.make_async_copy(sd_hbm.at[0, pl.ds(0, 2 * NSUB)], sd[q],
                              semI[q]).wait()
        pltpu.make_async_copy(a0_hbm.at[pl.ds(0, 3 * CH)], attr[q],
                              semI[q]).wait()

    def fire_gather(g, p, q):
        for j in range(NSUB):
            pltpu.async_copy(h2_hbm.at[sd[q].at[j]],
                             hrows[p].at[pl.ds(j * 128, 128)], semG[p])

    def wait_gather(p):
        pltpu.make_async_copy(h2_hbm.at[pl.ds(0, CH)], hrows[p],
                              semG[p]).wait()

    def compute(p, q):
        hv = hrows[p]
        av = attr[q]

        def _grp(g16):
            b0 = g16 * L
            a0v = av[pl.ds(b0, L)]
            a1v = av[pl.ds(CH + b0, L)]
            a2v = av[pl.ds(2 * CH + b0, L)]
            for i in range(L):
                b = b0 + i
                p01 = a0v[i] * ew0 + a1v[i] * ew1
                p2b = a2v[i] * ew2 + ebv
                hb = hv[b, :] + p01
                hv[b, :] = jnp.maximum(hb + p2b, 0.0)
        plsc.parallel_loop(0, CH // L)(_grp)

    def fire_scatter(p, q):
        for j in range(NSUB):
            pltpu.async_copy(hrows[p].at[pl.ds(j * 128, 128)],
                             acc_sh.at[sd[q].at[NSUB + j]], semS[p],
                             add=True)

    def wait_scatter(p):
        pltpu.make_async_copy(hrows[p], acc_sh.at[pl.ds(0, CH)],
                              semS[p]).wait()

    def phase(cidx, u, do_loads=True):
        wait_scatter(u % 2)
        wait_loads((u + 2) % 4)
        fire_gather(cidx, u % 2, (u + 2) % 4)
        wait_gather((u + 1) % 2)
        compute((u + 1) % 2, (u + 1) % 4)
        fire_scatter((u + 1) % 2, (u + 1) % 4)
        if do_loads:
            fire_loads(cidx + 2, u % 4)

    for q in range(4):
        fire_loads(q, q)
    wait_loads(0)
    fire_gather(0, 0, 0)
    wait_loads(1)
    fire_gather(1, 1, 1)
    wait_gather(0)
    compute(0, 0)
    fire_scatter(0, 0)

    def _body(k, _):
        cbase = 4 * k + 2
        for u in range(4):
            phase(cbase + u, u)
        return ()
    lax.fori_loop(0, (CHUNKS - 4) // 4, _body, ())

    phase(CHUNKS - 2, 0, do_loads=False)
    phase(CHUNKS - 1, 1, do_loads=False)
    wait_gather(1)
    compute(1, 3)
    fire_scatter(1, 3)
    wait_scatter(0)
    wait_scatter(1)
    plsc.subcore_barrier()

    w0 = s * ZPT
    nwb = ZPT // CH + (1 if ZPT % CH else 0)

    def _wb(k):
        off = k * CH
        size = min(CH, ZPT - off)
        return off, size

    def fire_rd(k):
        off, size = _wb(k)
        pltpu.async_copy(acc_sh.at[pl.ds(w0 + off, size)],
                         hrows[k % 2].at[pl.ds(0, size)], semG[k % 2])

    def wait_rd(k):
        off, size = _wb(k)
        pltpu.make_async_copy(acc_sh.at[pl.ds(0, size)],
                              hrows[k % 2].at[pl.ds(0, size)],
                              semG[k % 2]).wait()

    def fire_wr(k):
        off, size = _wb(k)
        pltpu.async_copy(hrows[k % 2].at[pl.ds(0, size)],
                         agg_hbm.at[c, pl.ds(w0 + off, size)], semS[k % 2])

    def wait_wr(k):
        off, size = _wb(k)
        pltpu.make_async_copy(hrows[k % 2].at[pl.ds(0, size)],
                              agg_hbm.at[c, pl.ds(0, size)],
                              semS[k % 2]).wait()

    fire_rd(0)
    for k in range(nwb):
        if k + 1 < nwb:
            if k - 1 >= 0:
                wait_wr(k - 1)
            fire_rd(k + 1)
        wait_rd(k)
        fire_wr(k)
    for k in (nwb - 2, nwb - 1):
        if k >= 0:
            wait_wr(k)


@functools.partial(
    pl.kernel,
    out_type=jax.ShapeDtypeStruct((NC, PG_R, PW), jnp.float32),
    mesh=_mesh,
    compiler_params=_sc_params,
    scratch_types=[
        pltpu.VMEM_SHARED((PG_R, PW), jnp.float32),
        pltpu.VMEM((128, 32), jnp.float32),
        pltpu.VMEM((128, PW), jnp.float32),
        pltpu.VMEM((PCH, 128), jnp.int32),
        pltpu.VMEM((2, 128), jnp.float32),
    ],
)
def _pool_pass(y_hbm, bidx_hbm, stats_hbm, pooled_hbm,
               acc_sh, yv, mbuf, bidx_v, ssv):
    c = lax.axis_index("c")
    s = lax.axis_index("s")
    w = s * NC + c
    zpt = PG_R // NS

    pltpu.sync_copy(stats_hbm.at[pl.ds(2, 2)], ssv)
    sc0 = ssv[0, 0:16]
    sc1 = ssv[0, 16:32]
    sh0 = ssv[1, 0:16]
    sh1 = ssv[1, 16:32]

    zv = jnp.zeros((L,), jnp.float32)
    def _zrow(b, _):
        mbuf[b, 0:16] = zv
        mbuf[b, 16:32] = zv
        mbuf[b, 32:48] = zv
        return ()
    lax.fori_loop(0, 128, _zrow, (), unroll=4)
    pltpu.sync_copy(mbuf.at[pl.ds(0, zpt)], acc_sh.at[pl.ds(s * zpt, zpt)])

    cvec = jnp.where(lax.iota(jnp.int32, L) == 0, 1.0, 0.0).astype(jnp.float32)
    def _crow(b, _):
        mbuf[b, 32:48] = cvec
        return ()
    lax.fori_loop(0, 128, _crow, ())
    plsc.subcore_barrier()

    pltpu.sync_copy(bidx_hbm.at[pl.ds(w * PCH, PCH)], bidx_v)

    def _chunk(t, _):
        r0 = w * PB + t * 128
        pltpu.sync_copy(y_hbm.at[pl.ds(r0, 128)], yv)

        def _row(b, _):
            mbuf[b, 0:16] = jnp.maximum(yv[b, 0:16] * sc0 + sh0, 0.0)
            mbuf[b, 16:32] = jnp.maximum(yv[b, 16:32] * sc1 + sh1, 0.0)
            return ()
        lax.fori_loop(0, 128, _row, (), unroll=4)

        pltpu.sync_copy(mbuf, acc_sh.at[bidx_v.at[t]], add=True)
        return ()

    lax.fori_loop(0, PCH, _chunk, ())
    plsc.subcore_barrier()

    pltpu.sync_copy(acc_sh.at[pl.ds(s * zpt, zpt)], mbuf.at[pl.ds(0, zpt)])
    pltpu.sync_copy(mbuf.at[pl.ds(0, zpt)],
                    pooled_hbm.at[c, pl.ds(s * zpt, zpt)])


_BN1 = 2048
_NBLK = NP // _BN1


def _k1_body(x_ref, w_ref, b_ref, out_ref):
    y = jnp.dot(x_ref[...], w_ref[...], preferred_element_type=jnp.float32,
                precision=lax.Precision.HIGHEST) + b_ref[...]
    out_ref[0] = y[:, :16]
    out_ref[1] = y[:, 16:]


def _node_mlp(xp, node_w, node_b):
    return pl.pallas_call(
        _k1_body,
        grid=(_NBLK,),
        in_specs=[
            pl.BlockSpec((_BN1, 14), lambda i: (i, 0)),
            pl.BlockSpec((14, 32), lambda i: (0, 0)),
            pl.BlockSpec((32,), lambda i: (0,)),
        ],
        out_specs=pl.BlockSpec((2, _BN1, 16), lambda i: (0, i, 0)),
        out_shape=jax.ShapeDtypeStruct((2, NP, 16), jnp.float32),
    )(xp, node_w, node_b)


def _k2_body(h_ref, agg_ref, w1_ref, b1_ref, w2_ref, b2_ref, g_ref, bb_ref,
             y_ref, st_ref):
    i = pl.program_id(0)
    h = jnp.concatenate([h_ref[0], h_ref[1]], axis=1)
    a = jnp.concatenate([agg_ref[0], agg_ref[1]], axis=1)
    z = h + a
    t = jnp.maximum(jnp.dot(z, w1_ref[...], preferred_element_type=jnp.float32,
                            precision=lax.Precision.HIGHEST) + b1_ref[...], 0.0)
    y = jnp.dot(t, w2_ref[...], preferred_element_type=jnp.float32,
                precision=lax.Precision.HIGHEST) + b2_ref[...]
    y_ref[...] = y

    rows = lax.broadcasted_iota(jnp.int32, (_BN1, 1), 0) + i * _BN1
    ym = jnp.where(rows < N, y, 0.0)

    @pl.when(i == 0)
    def _():
        st_ref[...] = jnp.zeros((8, 128), jnp.float32)

    ssum = jnp.sum(ym, axis=0)
    ssq = jnp.sum(ym * ym, axis=0)
    delta = jnp.pad(jnp.stack([ssum, ssq], axis=0), ((0, 6), (0, 96)))
    st_ref[...] += delta

    @pl.when(i == _NBLK - 1)
    def _():
        st = st_ref[...]
        mu = st[0, :32] / N
        var = st[1, :32] / N - mu * mu
        scale = g_ref[...] * lax.rsqrt(var + BN_EPS)
        shift = bb_ref[...] - mu * scale
        st_ref[pl.ds(2, 2), :] = jnp.pad(
            jnp.stack([scale, shift], axis=0), ((0, 0), (0, 96)))


def _gine_mlp(h2, agg, w1, b1, w2, b2, g, bb):
    return pl.pallas_call(
        _k2_body,
        grid=(_NBLK,),
        in_specs=[
            pl.BlockSpec((2, _BN1, 16), lambda i: (0, i, 0)),
            pl.BlockSpec((2, _BN1, 16), lambda i: (0, i, 0)),
            pl.BlockSpec((32, 75), lambda i: (0, 0)),
            pl.BlockSpec((75,), lambda i: (0,)),
            pl.BlockSpec((75, 32), lambda i: (0, 0)),
            pl.BlockSpec((32,), lambda i: (0,)),
            pl.BlockSpec((32,), lambda i: (0,)),
            pl.BlockSpec((32,), lambda i: (0,)),
        ],
        out_specs=[
            pl.BlockSpec((_BN1, 32), lambda i: (i, 0)),
            pl.BlockSpec((8, 128), lambda i: (0, 0)),
        ],
        out_shape=[
            jax.ShapeDtypeStruct((NP, 32), jnp.float32),
            jax.ShapeDtypeStruct((8, 128), jnp.float32),
        ],
    )(h2, agg, w1, b1, w2, b2, g, bb)


def _k4_body(y_ref, st_ref, out_ref):
    sc = st_ref[2, :32]
    sh = st_ref[3, :32]
    hh = jnp.maximum(y_ref[...] * sc + sh, 0.0)
    out_ref[0] = hh[:, :16]
    out_ref[1] = hh[:, 16:]


def _bn_apply(y, stats):
    return pl.pallas_call(
        _k4_body,
        grid=(_NBLK,),
        in_specs=[
            pl.BlockSpec((_BN1, 32), lambda i: (i, 0)),
            pl.BlockSpec((8, 128), lambda i: (0, 0)),
        ],
        out_specs=pl.BlockSpec((2, _BN1, 16), lambda i: (0, i, 0)),
        out_shape=jax.ShapeDtypeStruct((2, NP, 16), jnp.float32),
    )(y, stats)


def _k5_body(p_ref, w1_ref, b1_ref, w2_ref, b2_ref, out_ref):
    sall = p_ref[0] + p_ref[1]
    sums = sall[:G, :32]
    cnt = sall[:G, 32:33]
    gx = sums / jnp.maximum(cnt, 1.0)
    t = jnp.maximum(jnp.dot(gx, w1_ref[...], preferred_element_type=jnp.float32,
                            precision=lax.Precision.HIGHEST) + b1_ref[...], 0.0)
    out_ref[...] = jnp.dot(t, w2_ref[...], preferred_element_type=jnp.float32,
                           precision=lax.Precision.HIGHEST) + b2_ref[...]


def _head(pooled, l1_w, l1_b, l2_w, l2_b):
    return pl.pallas_call(
        _k5_body,
        out_shape=jax.ShapeDtypeStruct((G, 2), jnp.float32),
    )(pooled, l1_w, l1_b, l2_w, l2_b)


def kernel(x, edge_index, edge_attr, batch,
           node_w, node_b, edge_w, edge_b,
           c0_w1, c0_b1, c0_w2, c0_b2, bn0_g, bn0_b,
           c1_w1, c1_b1, c1_w2, c1_b2, bn1_g, bn1_b,
           l1_w, l1_b, l2_w, l2_b):
    xp = jnp.pad(x, ((0, NP - N), (0, 0)))
    batchp = jnp.pad(batch, (0, NP - N), constant_values=G).reshape(
        NP // 128, 128)

    pad_e = EP - E
    ar = jnp.arange(pad_e, dtype=jnp.int32) % 256
    srcp = jnp.concatenate([edge_index[0], ar])
    src2 = jnp.stack([srcp, srcp + NP])
    dstp = jnp.concatenate([edge_index[1], N + ar])
    tc_ = EP // CH
    srcr = src2.reshape(2, tc_, NSUB, 128)
    dstr = jnp.broadcast_to(dstp.reshape(1, tc_, NSUB, 128),
                            (2, tc_, NSUB, 128))
    sd2 = jnp.concatenate([srcr, dstr], axis=2).reshape(
        2, tc_ * 2 * NSUB, 128)
    attrp = jnp.pad(edge_attr, ((0, pad_e), (0, 0)))
    a0p = attrp[:, 0]
    a1p = attrp[:, 1]
    a2p = attrp[:, 2]
    ew_h = edge_w.reshape(3, 2, 16).transpose(1, 0, 2)
    ewb = jnp.concatenate([ew_h, edge_b.reshape(2, 1, 16)], axis=1)

    h2 = _node_mlp(xp, node_w, node_b)
    agg0 = _edge_pass(h2.reshape(2 * NP, 16), sd2, a0p, a1p, a2p, ewb)
    y0, stats0 = _gine_mlp(h2, agg0, c0_w1, c0_b1, c0_w2, c0_b2, bn0_g, bn0_b)
    h2b = _bn_apply(y0, stats0)
    agg1 = _edge_pass(h2b.reshape(2 * NP, 16), sd2, a0p, a1p, a2p, ewb)
    y1, stats1 = _gine_mlp(h2b, agg1, c1_w1, c1_b1, c1_w2, c1_b2, bn1_g, bn1_b)
    pooled = _pool_pass(y1, batchp, stats1)
    return _head(pooled, l1_w, l1_b, l2_w, l2_b)
